# baseline (speedup 1.0000x reference)
import functools

import jax
import jax.numpy as jnp
from jax import lax
from jax.experimental import pallas as pl
from jax.experimental.pallas import tpu as pltpu
from jax.experimental.pallas import tpu_sc as plsc

N = 10000
NP = 10240
HALF = NP // 2
E = 320000
G = 64
D = 128
NC, NS = 2, 16
NT = NC * NS
CH = 80
NCH = 126
NCH_TOT = NT * NCH
EPT = NCH * CH
EP = NCH_TOT * CH
NBUF = 3
NCHMAX = 138
TABR = HALF + 16
PADIDX = HALF
RB = 1024
NBLK = NP // RB

_mesh = plsc.VectorSubcoreMesh(core_axis_name="c", subcore_axis_name="s")
_sc_params = pltpu.CompilerParams(needs_layout_passes=False)


@functools.partial(
    pl.kernel,
    out_type=jax.ShapeDtypeStruct((NT, NP), jnp.float32),
    mesh=_mesh,
    scratch_types=[
        pltpu.VMEM((EPT,), jnp.int32),
        pltpu.VMEM((NP,), jnp.float32),
    ],
    compiler_params=_sc_params,
)
def _deg(dst_flat, out, dv, part):
    c = lax.axis_index("c")
    s = lax.axis_index("s")
    wid = c * NS + s
    pltpu.sync_copy(dst_flat.at[wid], dv)

    def zero_body(i, carry):
        part[pl.ds(i * 16, 16)] = jnp.zeros((16,), jnp.float32)
        return carry
    lax.fori_loop(0, NP // 16, zero_body, 0)

    ones = jnp.ones((16,), jnp.float32)

    def add_body(i, carry):
        idx = dv[pl.ds(i * 16, 16)]
        plsc.addupdate_scatter(part, [idx], ones)
        return carry
    lax.fori_loop(0, EPT // 16, add_body, 0)

    pltpu.sync_copy(part, out.at[wid])


@functools.partial(
    pl.kernel,
    out_type=[
        jax.ShapeDtypeStruct((NT, NCHMAX, CH), jnp.int32),
        jax.ShapeDtypeStruct((NT, NCHMAX, CH), jnp.int32),
        jax.ShapeDtypeStruct((NT, 16), jnp.int32),
    ],
    mesh=_mesh,
    scratch_types=[
        pltpu.VMEM((NBUF, CH), jnp.int32),
        pltpu.VMEM((NBUF, CH), jnp.int32),
        pltpu.VMEM((NCHMAX, CH), jnp.int32),
        pltpu.VMEM((NCHMAX, CH), jnp.int32),
        pltpu.VMEM((16,), jnp.int32),
        pltpu.SemaphoreType.DMA((NBUF,)),
    ],
    compiler_params=_sc_params,
)
def _part(srcr, dstr, psrc, pdst, cnts, si, di, pv, qv, cv, isem):
    c = lax.axis_index("c")
    s = lax.axis_index("s")
    wid = c * NS + s
    cbase = wid * NCH

    def pre2(i, carry):
        for k in range(CH // 16):
            pv[i, pl.ds(k * 16, 16)] = jnp.full((16,), PADIDX, jnp.int32)
            qv[i, pl.ds(k * 16, 16)] = jnp.full((16,), PADIDX, jnp.int32)
        return carry
    lax.fori_loop(0, NCHMAX, pre2, 0)

    def ring(body_fn, carry0):
        for b in range(NBUF):
            pltpu.async_copy(srcr.at[cbase + b], si.at[b], isem.at[b])
            pltpu.async_copy(dstr.at[cbase + b], di.at[b], isem.at[b])

        def group(g, carry):
            for b in range(NBUF):
                j = cbase + g * NBUF + b
                pltpu.make_async_copy(srcr.at[j], si.at[b],
                                      isem.at[b]).wait()
                pltpu.make_async_copy(dstr.at[j], di.at[b],
                                      isem.at[b]).wait()
                carry = body_fn(b, carry)
                j2 = g * NBUF + b + NBUF

                @pl.when(j2 < NCH)
                def _():
                    pltpu.async_copy(srcr.at[cbase + j2], si.at[b],
                                     isem.at[b])
                    pltpu.async_copy(dstr.at[cbase + j2], di.at[b],
                                     isem.at[b])
            return carry
        return lax.fori_loop(0, NCH // NBUF, group, carry0)

    def count_chunk(b, carry):
        def vec(v, carry):
            sv = si[b, pl.ds(v * 16, 16)]
            dv2 = di[b, pl.ds(v * 16, 16)]
            key = (sv >= HALF).astype(jnp.int32) * 2 \
                + (dv2 >= HALF).astype(jnp.int32)
            c0, c1, c2, c3 = carry
            c0 = c0 + jnp.sum((key == 0).astype(jnp.int32))
            c1 = c1 + jnp.sum((key == 1).astype(jnp.int32))
            c2 = c2 + jnp.sum((key == 2).astype(jnp.int32))
            c3 = c3 + jnp.sum((key == 3).astype(jnp.int32))
            return (c0, c1, c2, c3)
        return lax.fori_loop(0, CH // 16, vec, carry)

    cnt = ring(count_chunk, (jnp.int32(0),) * 4)

    grp_e = NBUF * CH
    nchk = [((ck + grp_e - 1) // grp_e) * NBUF for ck in cnt]
    st = [jnp.int32(0)] * 4
    for k in range(1, 4):
        st[k] = st[k - 1] + nchk[k - 1]

    def part_chunk(b, carry):
        def vec(v, offs):
            sv = si[b, pl.ds(v * 16, 16)]
            dv2 = di[b, pl.ds(v * 16, 16)]
            amask = sv >= HALF
            bmask = dv2 >= HALF
            key = amask.astype(jnp.int32) * 2 + bmask.astype(jnp.int32)
            sreb = sv - amask.astype(jnp.int32) * HALF
            dreb = dv2 - bmask.astype(jnp.int32) * HALF
            new = []
            for k in range(4):
                mk = key == k
                mi = mk.astype(jnp.int32)
                cs = plsc.cumsum(mi)
                idx = offs[k] + cs - 1
                q = idx // CH
                r = idx - q * CH
                plsc.store_scatter(pv, [q, r], sreb, mask=mk)
                plsc.store_scatter(qv, [q, r], dreb, mask=mk)
                new.append(offs[k] + jnp.sum(mi))
            return tuple(new)
        return lax.fori_loop(0, CH // 16, vec, carry)

    ring(part_chunk, tuple(stk * CH for stk in st))

    lane = lax.iota(jnp.int32, 16)
    cvvec = jnp.full((16,), 0, jnp.int32)
    for k in range(4):
        cvvec = jnp.where(lane == k, jnp.full((16,), 1, jnp.int32) * nchk[k],
                          cvvec)
    cv[pl.ds(0, 16)] = cvvec

    pltpu.sync_copy(pv, psrc.at[wid])
    pltpu.sync_copy(qv, pdst.at[wid])
    pltpu.sync_copy(cv, cnts.at[wid])


RSTG = HALF // NS
RZ = TABR // NS


@functools.partial(
    pl.kernel,
    out_type=jax.ShapeDtypeStruct((NC, NP, D), jnp.float32),
    mesh=_mesh,
    scratch_types=[
        pltpu.VMEM((NBUF, CH), jnp.int32),
        pltpu.VMEM((NBUF, CH), jnp.int32),
        pltpu.VMEM((CH, D), jnp.float32),
        pltpu.VMEM((CH, D), jnp.float32),
        pltpu.VMEM((CH, D), jnp.float32),
        pltpu.VMEM((16,), jnp.int32),
        pltpu.SemaphoreType.DMA((NBUF,)),
        pltpu.SemaphoreType.DMA((NBUF,)),
        pltpu.SemaphoreType.DMA((NBUF,)),
        pltpu.VMEM_SHARED((TABR, D), jnp.float32),
        pltpu.VMEM_SHARED((TABR, D), jnp.float32),
    ],
    compiler_params=_sc_params,
)
def _prop(hp, psrc, pdst, cnts, out, si, di, b0, b1, b2, cv, isem, gsem,
          ssem, tab, acc):
    bufs = (b0, b1, b2)
    c = lax.axis_index("c")
    s = lax.axis_index("s")
    wid = c * NS + s

    pltpu.sync_copy(cnts.at[wid], cv)
    cvv = cv[pl.ds(0, 16)]
    nchk = [cvv[k] for k in range(4)]
    st = [jnp.int32(0)] * 4
    for k in range(1, 4):
        st[k] = st[k - 1] + nchk[k - 1]

    def zero_b0():
        def zrow(i, carry):
            for k in range(D // 16):
                b0[i, pl.ds(k * 16, 16)] = jnp.zeros((16,), jnp.float32)
            return carry
        lax.fori_loop(0, CH, zrow, 0)

    for b in range(2):
        zero_b0()
        for off in (0, CH, 2 * CH, 3 * CH, RZ - CH):
            pltpu.sync_copy(b0, acc.at[pl.ds(s * RZ + off, CH)])

        for a in range(2):
            k = 2 * a + b
            if a == 1:
                zero_b0()
            pltpu.sync_copy(hp.at[pl.ds(a * HALF + s * RSTG, RSTG)],
                            tab.at[pl.ds(s * RSTG, RSTG)])

            @pl.when(s == 0)
            def _():
                pltpu.sync_copy(b0.at[pl.ds(0, 16)], tab.at[pl.ds(HALF, 16)])
            plsc.subcore_barrier()

            for bb in range(NBUF):
                @pl.when(bb < nchk[k])
                def _():
                    cj = st[k] + bb
                    pltpu.async_copy(psrc.at[wid, cj], si.at[bb],
                                     isem.at[bb])
                    pltpu.async_copy(pdst.at[wid, cj], di.at[bb],
                                     isem.at[bb])

            def group(g, carry):
                for bb in range(NBUF):
                    cj = st[k] + g * NBUF + bb
                    pltpu.make_async_copy(psrc.at[wid, cj], si.at[bb],
                                          isem.at[bb]).wait()
                    pltpu.make_async_copy(pdst.at[wid, cj], di.at[bb],
                                          isem.at[bb]).wait()
                    pltpu.async_copy(tab.at[si.at[bb]], bufs[bb],
                                     gsem.at[bb])
                for bb in range(NBUF):
                    pltpu.make_async_copy(tab.at[si.at[bb]], bufs[bb],
                                          gsem.at[bb]).wait()
                    pltpu.async_copy(bufs[bb], acc.at[di.at[bb]],
                                     ssem.at[bb], add=True)
                for bb in range(NBUF):
                    j2 = g * NBUF + bb + NBUF
                    pltpu.make_async_copy(bufs[bb], acc.at[di.at[bb]],
                                          ssem.at[bb]).wait()

                    @pl.when(j2 < nchk[k])
                    def _():
                        cj2 = st[k] + j2
                        pltpu.async_copy(psrc.at[wid, cj2], si.at[bb],
                                         isem.at[bb])
                        pltpu.async_copy(pdst.at[wid, cj2], di.at[bb],
                                         isem.at[bb])
                return carry
            lax.fori_loop(0, nchk[k] // NBUF, group, 0)

            plsc.subcore_barrier()

        pltpu.sync_copy(
            acc.at[pl.ds(s * RSTG, RSTG)],
            out.at[c, pl.ds(b * HALF + s * RSTG, RSTG)])
        plsc.subcore_barrier()


def _tc_a_body(degp_ref, x_ref, w1_ref, h1p_ref, dis_ref):
    ones = jnp.ones((NT, 1), jnp.float32)
    deg = lax.dot_general(degp_ref[...], ones, (((0,), (0,)), ((), ()))) + 1.0
    dis = lax.rsqrt(deg)
    h = jnp.dot(x_ref[...], w1_ref[...], preferred_element_type=jnp.float32)
    h1p_ref[...] = h * dis
    dis_ref[...] = jnp.broadcast_to(dis, (RB, D))


_tc_a = pl.pallas_call(
    _tc_a_body,
    grid=(NBLK,),
    in_specs=[
        pl.BlockSpec((NT, RB), lambda i: (0, i)),
        pl.BlockSpec((RB, D), lambda i: (i, 0)),
        pl.BlockSpec((D, D), lambda i: (0, 0)),
    ],
    out_specs=[
        pl.BlockSpec((RB, D), lambda i: (i, 0)),
        pl.BlockSpec((RB, D), lambda i: (i, 0)),
    ],
    out_shape=[
        jax.ShapeDtypeStruct((NP, D), jnp.float32),
        jax.ShapeDtypeStruct((NP, D), jnp.float32),
    ],
)


def _tc_b_body(acc_ref, h1p_ref, dis_ref, b1_ref, w2_ref, out_ref):
    ssum = jnp.sum(acc_ref[...], axis=0) + h1p_ref[...]
    h = jnp.maximum(dis_ref[...] * ssum + b1_ref[...], 0.0)
    out_ref[...] = dis_ref[...] * jnp.dot(
        h, w2_ref[...], preferred_element_type=jnp.float32)


_tc_b = pl.pallas_call(
    _tc_b_body,
    grid=(NBLK,),
    in_specs=[
        pl.BlockSpec((NC, RB, D), lambda i: (0, i, 0)),
        pl.BlockSpec((RB, D), lambda i: (i, 0)),
        pl.BlockSpec((RB, D), lambda i: (i, 0)),
        pl.BlockSpec((1, D), lambda i: (0, 0)),
        pl.BlockSpec((D, D), lambda i: (0, 0)),
    ],
    out_specs=pl.BlockSpec((RB, D), lambda i: (i, 0)),
    out_shape=jax.ShapeDtypeStruct((NP, D), jnp.float32),
)


def _tc_c_body(acc_ref, h2p_ref, dis_ref, b2_ref, batch_ref, out_ref,
               cnt_ref):
    i = pl.program_id(0)
    conv = dis_ref[...] * (jnp.sum(acc_ref[...], axis=0) + h2p_ref[...]) \
        + b2_ref[...]
    b = batch_ref[0]
    gids = lax.broadcasted_iota(jnp.int32, (G, RB), 0)
    onehot = (b == gids).astype(jnp.float32)
    psum = jnp.dot(onehot, conv, preferred_element_type=jnp.float32)
    pcnt = jnp.sum(onehot, axis=1, keepdims=True)

    @pl.when(i == 0)
    def _():
        out_ref[...] = jnp.zeros((G, D), jnp.float32)
        cnt_ref[...] = jnp.zeros((G, 1), jnp.float32)

    out_ref[...] += psum
    cnt_ref[...] += pcnt

    @pl.when(i == NBLK - 1)
    def _():
        out_ref[...] = out_ref[...] / jnp.maximum(cnt_ref[...], 1.0)


_tc_c = pl.pallas_call(
    _tc_c_body,
    grid=(NBLK,),
    in_specs=[
        pl.BlockSpec((NC, RB, D), lambda i: (0, i, 0)),
        pl.BlockSpec((RB, D), lambda i: (i, 0)),
        pl.BlockSpec((RB, D), lambda i: (i, 0)),
        pl.BlockSpec((1, D), lambda i: (0, 0)),
        pl.BlockSpec((1, 1, RB), lambda i: (i, 0, 0)),
    ],
    out_specs=pl.BlockSpec((G, D), lambda i: (0, 0)),
    out_shape=jax.ShapeDtypeStruct((G, D), jnp.float32),
    scratch_shapes=[pltpu.VMEM((G, 1), jnp.float32)],
)


def kernel(x, edge_index, batch, W1, b1, W2, b2):
    src = edge_index[0]
    dst = edge_index[1]
    pad_e = jnp.full((EP - E,), N, jnp.int32)
    srcr = jnp.concatenate([src, pad_e]).reshape(NCH_TOT, CH)
    dstp = jnp.concatenate([dst, pad_e])
    dstr = dstp.reshape(NCH_TOT, CH)
    dst_flat = dstp.reshape(NT, EPT)
    x_pad = jnp.pad(x, ((0, NP - N), (0, 0)))
    batch_pad = jnp.concatenate(
        [batch, jnp.full((NP - N,), G, jnp.int32)]).reshape(NBLK, 1, RB)

    psrc, pdst, cnts = _part(srcr, dstr)
    degp = _deg(dst_flat)
    h1p, disb = _tc_a(degp, x_pad, W1)
    acc1 = _prop(h1p, psrc, pdst, cnts)
    h2p = _tc_b(acc1, h1p, disb, b1.reshape(1, D), W2)
    acc2 = _prop(h2p, psrc, pdst, cnts)
    pooled = _tc_c(acc2, h2p, disb, b2.reshape(1, D), batch_pad)
    return pooled


# speedup vs baseline: 1.1863x; 1.1863x over previous
import functools

import jax
import jax.numpy as jnp
from jax import lax
from jax.experimental import pallas as pl
from jax.experimental.pallas import tpu as pltpu
from jax.experimental.pallas import tpu_sc as plsc

N = 10000
NP = 10240
HALF = NP // 2
E = 320000
G = 64
D = 128
NC, NS = 2, 16
NT = NC * NS
CH = 80
NCH = 126
NCH_TOT = NT * NCH
EPT = NCH * CH
EP = NCH_TOT * CH
NBUF = 3
NCHMAX = 138
TABR = HALF + 16
PADIDX = HALF
RB = 1024
NBLK = NP // RB

_mesh = plsc.VectorSubcoreMesh(core_axis_name="c", subcore_axis_name="s")
_sc_params = pltpu.CompilerParams(needs_layout_passes=False)


@functools.partial(
    pl.kernel,
    out_type=jax.ShapeDtypeStruct((NT, NP), jnp.float32),
    mesh=_mesh,
    scratch_types=[
        pltpu.VMEM((EPT,), jnp.int32),
        pltpu.VMEM((NP,), jnp.float32),
    ],
    compiler_params=_sc_params,
)
def _deg(dst_flat, out, dv, part):
    c = lax.axis_index("c")
    s = lax.axis_index("s")
    wid = c * NS + s
    pltpu.sync_copy(dst_flat.at[wid], dv)

    def zero_body(i, carry):
        part[pl.ds(i * 16, 16)] = jnp.zeros((16,), jnp.float32)
        return carry
    lax.fori_loop(0, NP // 16, zero_body, 0)

    ones = jnp.ones((16,), jnp.float32)

    def add_body(i, carry):
        idx = dv[pl.ds(i * 16, 16)]
        plsc.addupdate_scatter(part, [idx], ones)
        return carry
    lax.fori_loop(0, EPT // 16, add_body, 0)

    pltpu.sync_copy(part, out.at[wid])


@functools.partial(
    pl.kernel,
    out_type=[
        jax.ShapeDtypeStruct((NT, NCHMAX * CH), jnp.int32),
        jax.ShapeDtypeStruct((NT, NCHMAX * CH), jnp.int32),
        jax.ShapeDtypeStruct((NT, 16), jnp.int32),
    ],
    mesh=_mesh,
    scratch_types=[
        pltpu.VMEM((NBUF, CH), jnp.int32),
        pltpu.VMEM((NBUF, CH), jnp.int32),
        pltpu.VMEM((NCHMAX * CH,), jnp.int32),
        pltpu.VMEM((NCHMAX * CH,), jnp.int32),
        pltpu.VMEM((16,), jnp.int32),
        pltpu.SemaphoreType.DMA((NBUF,)),
    ],
    compiler_params=_sc_params,
)
def _part(srcr, dstr, psrc, pdst, cnts, si, di, pvf, qvf, cv, isem):
    c = lax.axis_index("c")
    s = lax.axis_index("s")
    wid = c * NS + s
    cbase = wid * NCH

    def pre2(i, carry):
        pvf[pl.ds(i * 16, 16)] = jnp.full((16,), PADIDX, jnp.int32)
        qvf[pl.ds(i * 16, 16)] = jnp.full((16,), PADIDX, jnp.int32)
        return carry
    lax.fori_loop(0, NCHMAX * CH // 16, pre2, 0)

    def ring(body_fn, carry0):
        for b in range(NBUF):
            pltpu.async_copy(srcr.at[cbase + b], si.at[b], isem.at[b])
            pltpu.async_copy(dstr.at[cbase + b], di.at[b], isem.at[b])

        def group(g, carry):
            for b in range(NBUF):
                j = cbase + g * NBUF + b
                pltpu.make_async_copy(srcr.at[j], si.at[b],
                                      isem.at[b]).wait()
                pltpu.make_async_copy(dstr.at[j], di.at[b],
                                      isem.at[b]).wait()
                carry = body_fn(b, carry)
                j2 = g * NBUF + b + NBUF

                @pl.when(j2 < NCH)
                def _():
                    pltpu.async_copy(srcr.at[cbase + j2], si.at[b],
                                     isem.at[b])
                    pltpu.async_copy(dstr.at[cbase + j2], di.at[b],
                                     isem.at[b])
            return carry
        return lax.fori_loop(0, NCH // NBUF, group, carry0)

    def count_chunk(b, carry):
        def vec(v, carry):
            sv = si[b, pl.ds(v * 16, 16)]
            dv2 = di[b, pl.ds(v * 16, 16)]
            key = (sv >= HALF).astype(jnp.int32) * 2 \
                + (dv2 >= HALF).astype(jnp.int32)
            return tuple(
                carry[k] + plsc.all_reduce_population_count(key == k)
                for k in range(4))
        return lax.fori_loop(0, CH // 16, vec, carry)

    czero = jnp.zeros((16,), jnp.int32)
    cntv = ring(count_chunk, (czero,) * 4)
    cnt = [cntv[k][0] for k in range(4)]

    grp_e = NBUF * CH
    nchk = [((ck + grp_e - 1) // grp_e) * NBUF for ck in cnt]
    st = [jnp.int32(0)] * 4
    for k in range(1, 4):
        st[k] = st[k - 1] + nchk[k - 1]

    def part_chunk(b, carry):
        def vec(v, offs):
            sv = si[b, pl.ds(v * 16, 16)]
            dv2 = di[b, pl.ds(v * 16, 16)]
            amask = sv >= HALF
            bmask = dv2 >= HALF
            key = amask.astype(jnp.int32) * 2 + bmask.astype(jnp.int32)
            sreb = sv - amask.astype(jnp.int32) * HALF
            dreb = dv2 - bmask.astype(jnp.int32) * HALF
            new = []
            for k in range(4):
                mk = key == k
                o = offs[k][0]
                plsc.store_compressed(pvf.at[pl.ds(o, 16)], sreb, mask=mk)
                plsc.store_compressed(qvf.at[pl.ds(o, 16)], dreb, mask=mk)
                new.append(offs[k] + plsc.all_reduce_population_count(mk))
            return tuple(new)
        return lax.fori_loop(0, CH // 16, vec, carry)

    ring(part_chunk, tuple(jnp.full((16,), stk * CH, jnp.int32)
                           for stk in st))

    lane = lax.iota(jnp.int32, 16)
    cvvec = jnp.full((16,), 0, jnp.int32)
    for k in range(4):
        cvvec = jnp.where(lane == k, jnp.full((16,), 1, jnp.int32) * nchk[k],
                          cvvec)
    cv[pl.ds(0, 16)] = cvvec

    pltpu.sync_copy(pvf, psrc.at[wid])
    pltpu.sync_copy(qvf, pdst.at[wid])
    pltpu.sync_copy(cv, cnts.at[wid])


RSTG = HALF // NS
RZ = TABR // NS


@functools.partial(
    pl.kernel,
    out_type=jax.ShapeDtypeStruct((NC, NP, D), jnp.float32),
    mesh=_mesh,
    scratch_types=[
        pltpu.VMEM((NBUF, CH), jnp.int32),
        pltpu.VMEM((NBUF, CH), jnp.int32),
        pltpu.VMEM((CH, D), jnp.float32),
        pltpu.VMEM((CH, D), jnp.float32),
        pltpu.VMEM((CH, D), jnp.float32),
        pltpu.VMEM((16,), jnp.int32),
        pltpu.SemaphoreType.DMA((NBUF,)),
        pltpu.SemaphoreType.DMA((NBUF,)),
        pltpu.SemaphoreType.DMA((NBUF,)),
        pltpu.VMEM_SHARED((TABR, D), jnp.float32),
        pltpu.VMEM_SHARED((TABR, D), jnp.float32),
    ],
    compiler_params=_sc_params,
)
def _prop(hp, psrc, pdst, cnts, out, si, di, b0, b1, b2, cv, isem, gsem,
          ssem, tab, acc):
    bufs = (b0, b1, b2)
    c = lax.axis_index("c")
    s = lax.axis_index("s")
    wid = c * NS + s

    pltpu.sync_copy(cnts.at[wid], cv)
    cvv = cv[pl.ds(0, 16)]
    nchk = [cvv[k] for k in range(4)]
    st = [jnp.int32(0)] * 4
    for k in range(1, 4):
        st[k] = st[k - 1] + nchk[k - 1]

    def zero_b0():
        def zrow(i, carry):
            for k in range(D // 16):
                b0[i, pl.ds(k * 16, 16)] = jnp.zeros((16,), jnp.float32)
            return carry
        lax.fori_loop(0, CH, zrow, 0)

    for b in range(2):
        zero_b0()
        for off in (0, CH, 2 * CH, 3 * CH, RZ - CH):
            pltpu.sync_copy(b0, acc.at[pl.ds(s * RZ + off, CH)])

        for a in range(2):
            k = 2 * a + b
            if a == 1:
                zero_b0()
            pltpu.sync_copy(hp.at[pl.ds(a * HALF + s * RSTG, RSTG)],
                            tab.at[pl.ds(s * RSTG, RSTG)])

            @pl.when(s == 0)
            def _():
                pltpu.sync_copy(b0.at[pl.ds(0, 16)], tab.at[pl.ds(HALF, 16)])
            plsc.subcore_barrier()

            for bb in range(NBUF):
                @pl.when(bb < nchk[k])
                def _():
                    cj = st[k] + bb
                    pltpu.async_copy(psrc.at[wid, cj], si.at[bb],
                                     isem.at[bb])
                    pltpu.async_copy(pdst.at[wid, cj], di.at[bb],
                                     isem.at[bb])

            def group(g, carry):
                for bb in range(NBUF):
                    cj = st[k] + g * NBUF + bb
                    pltpu.make_async_copy(psrc.at[wid, cj], si.at[bb],
                                          isem.at[bb]).wait()
                    pltpu.make_async_copy(pdst.at[wid, cj], di.at[bb],
                                          isem.at[bb]).wait()
                    pltpu.async_copy(tab.at[si.at[bb]], bufs[bb],
                                     gsem.at[bb])
                for bb in range(NBUF):
                    pltpu.make_async_copy(tab.at[si.at[bb]], bufs[bb],
                                          gsem.at[bb]).wait()
                    pltpu.async_copy(bufs[bb], acc.at[di.at[bb]],
                                     ssem.at[bb], add=True)
                for bb in range(NBUF):
                    j2 = g * NBUF + bb + NBUF
                    pltpu.make_async_copy(bufs[bb], acc.at[di.at[bb]],
                                          ssem.at[bb]).wait()

                    @pl.when(j2 < nchk[k])
                    def _():
                        cj2 = st[k] + j2
                        pltpu.async_copy(psrc.at[wid, cj2], si.at[bb],
                                         isem.at[bb])
                        pltpu.async_copy(pdst.at[wid, cj2], di.at[bb],
                                         isem.at[bb])
                return carry
            lax.fori_loop(0, nchk[k] // NBUF, group, 0)

            plsc.subcore_barrier()

        pltpu.sync_copy(
            acc.at[pl.ds(s * RSTG, RSTG)],
            out.at[c, pl.ds(b * HALF + s * RSTG, RSTG)])
        plsc.subcore_barrier()


def _tc_a_body(degp_ref, x_ref, w1_ref, h1p_ref, dis_ref):
    ones = jnp.ones((NT, 1), jnp.float32)
    deg = lax.dot_general(degp_ref[...], ones, (((0,), (0,)), ((), ()))) + 1.0
    dis = lax.rsqrt(deg)
    h = jnp.dot(x_ref[...], w1_ref[...], preferred_element_type=jnp.float32)
    h1p_ref[...] = h * dis
    dis_ref[...] = jnp.broadcast_to(dis, (RB, D))


_tc_a = pl.pallas_call(
    _tc_a_body,
    grid=(NBLK,),
    in_specs=[
        pl.BlockSpec((NT, RB), lambda i: (0, i)),
        pl.BlockSpec((RB, D), lambda i: (i, 0)),
        pl.BlockSpec((D, D), lambda i: (0, 0)),
    ],
    out_specs=[
        pl.BlockSpec((RB, D), lambda i: (i, 0)),
        pl.BlockSpec((RB, D), lambda i: (i, 0)),
    ],
    out_shape=[
        jax.ShapeDtypeStruct((NP, D), jnp.float32),
        jax.ShapeDtypeStruct((NP, D), jnp.float32),
    ],
)


def _tc_b_body(acc_ref, h1p_ref, dis_ref, b1_ref, w2_ref, out_ref):
    ssum = jnp.sum(acc_ref[...], axis=0) + h1p_ref[...]
    h = jnp.maximum(dis_ref[...] * ssum + b1_ref[...], 0.0)
    out_ref[...] = dis_ref[...] * jnp.dot(
        h, w2_ref[...], preferred_element_type=jnp.float32)


_tc_b = pl.pallas_call(
    _tc_b_body,
    grid=(NBLK,),
    in_specs=[
        pl.BlockSpec((NC, RB, D), lambda i: (0, i, 0)),
        pl.BlockSpec((RB, D), lambda i: (i, 0)),
        pl.BlockSpec((RB, D), lambda i: (i, 0)),
        pl.BlockSpec((1, D), lambda i: (0, 0)),
        pl.BlockSpec((D, D), lambda i: (0, 0)),
    ],
    out_specs=pl.BlockSpec((RB, D), lambda i: (i, 0)),
    out_shape=jax.ShapeDtypeStruct((NP, D), jnp.float32),
)


def _tc_c_body(acc_ref, h2p_ref, dis_ref, b2_ref, batch_ref, out_ref,
               cnt_ref):
    i = pl.program_id(0)
    conv = dis_ref[...] * (jnp.sum(acc_ref[...], axis=0) + h2p_ref[...]) \
        + b2_ref[...]
    b = batch_ref[0]
    gids = lax.broadcasted_iota(jnp.int32, (G, RB), 0)
    onehot = (b == gids).astype(jnp.float32)
    psum = jnp.dot(onehot, conv, preferred_element_type=jnp.float32)
    pcnt = jnp.sum(onehot, axis=1, keepdims=True)

    @pl.when(i == 0)
    def _():
        out_ref[...] = jnp.zeros((G, D), jnp.float32)
        cnt_ref[...] = jnp.zeros((G, 1), jnp.float32)

    out_ref[...] += psum
    cnt_ref[...] += pcnt

    @pl.when(i == NBLK - 1)
    def _():
        out_ref[...] = out_ref[...] / jnp.maximum(cnt_ref[...], 1.0)


_tc_c = pl.pallas_call(
    _tc_c_body,
    grid=(NBLK,),
    in_specs=[
        pl.BlockSpec((NC, RB, D), lambda i: (0, i, 0)),
        pl.BlockSpec((RB, D), lambda i: (i, 0)),
        pl.BlockSpec((RB, D), lambda i: (i, 0)),
        pl.BlockSpec((1, D), lambda i: (0, 0)),
        pl.BlockSpec((1, 1, RB), lambda i: (i, 0, 0)),
    ],
    out_specs=pl.BlockSpec((G, D), lambda i: (0, 0)),
    out_shape=jax.ShapeDtypeStruct((G, D), jnp.float32),
    scratch_shapes=[pltpu.VMEM((G, 1), jnp.float32)],
)


def kernel(x, edge_index, batch, W1, b1, W2, b2):
    src = edge_index[0]
    dst = edge_index[1]
    pad_e = jnp.full((EP - E,), N, jnp.int32)
    srcr = jnp.concatenate([src, pad_e]).reshape(NCH_TOT, CH)
    dstp = jnp.concatenate([dst, pad_e])
    dstr = dstp.reshape(NCH_TOT, CH)
    dst_flat = dstp.reshape(NT, EPT)
    x_pad = jnp.pad(x, ((0, NP - N), (0, 0)))
    batch_pad = jnp.concatenate(
        [batch, jnp.full((NP - N,), G, jnp.int32)]).reshape(NBLK, 1, RB)

    psrc, pdst, cnts = _part(srcr, dstr)
    psrc = psrc.reshape(NT, NCHMAX, CH)
    pdst = pdst.reshape(NT, NCHMAX, CH)
    degp = _deg(dst_flat)
    h1p, disb = _tc_a(degp, x_pad, W1)
    acc1 = _prop(h1p, psrc, pdst, cnts)
    h2p = _tc_b(acc1, h1p, disb, b1.reshape(1, D), W2)
    acc2 = _prop(h2p, psrc, pdst, cnts)
    pooled = _tc_c(acc2, h2p, disb, b2.reshape(1, D), batch_pad)
    return pooled
